# baseline (speedup 1.0000x reference)
import functools

import jax
import jax.numpy as jnp
import numpy as np
from jax import lax
from jax.experimental import pallas as pl
from jax.experimental.pallas import tpu as pltpu
from jax.experimental.pallas import tpu_sc as plsc

_USE_TEXT_QUERY = True
_USE_TEXT_CANDS = True
_N_CANDS = 5
_TEXT_QUESTION = 0
_TEXT_EMBEDDING = 1
_VISUAL_EMBEDDING = 3
_PREFIX_ROWS = 16


@functools.lru_cache(maxsize=None)
def _make_sc_prefix(D: int):
    mesh = plsc.ScalarSubcoreMesh(axis_name="c", num_cores=1)
    n_text = (1 if _USE_TEXT_QUERY else 0) + (
        _N_CANDS if _USE_TEXT_CANDS else 0
    )
    ids = [_TEXT_QUESTION] + [_TEXT_EMBEDDING] * (n_text - 1)
    ids += [_VISUAL_EMBEDDING] * (_PREFIX_ROWS - len(ids))

    @functools.partial(
        pl.kernel,
        mesh=mesh,
        out_type=jax.ShapeDtypeStruct((_PREFIX_ROWS, D), jnp.float32),
        scratch_types=[pltpu.SemaphoreType.DMA],
    )
    def k(table_hbm, out_hbm, sem):
        copies = [
            pltpu.async_copy(
                table_hbm.at[pl.ds(t, 1)], out_hbm.at[pl.ds(r, 1)], sem
            )
            for r, t in enumerate(ids)
        ]
        for c in copies:
            c.wait()

    return k


@functools.lru_cache(maxsize=None)
def _make_tc_fill(L: int, D: int):
    BLK = 1024
    assert L % BLK == 0 and BLK > _PREFIX_ROWS

    def body(table_ref, prefix_ref, out_ref):
        i = pl.program_id(0)
        row = table_ref[_VISUAL_EMBEDDING, :]
        out_ref[...] = jnp.broadcast_to(row[None, None, :], (BLK, 1, D))

        @pl.when(i == 0)
        def _splice():
            out_ref[0:_PREFIX_ROWS, :, :] = prefix_ref[...][:, None, :]

    return pl.pallas_call(
        body,
        grid=(L // BLK,),
        in_specs=[
            pl.BlockSpec((5, D), lambda i: (0, 0)),
            pl.BlockSpec((_PREFIX_ROWS, D), lambda i: (0, 0)),
        ],
        out_specs=pl.BlockSpec((BLK, 1, D), lambda i: (i, 0, 0)),
        out_shape=jax.ShapeDtypeStruct((L, 1, D), jnp.float32),
    )


def kernel(x, table):
    L, N, D = x.shape
    prefix = _make_sc_prefix(D)(table)
    return _make_tc_fill(L, D)(table, prefix)


# speedup vs baseline: 1.0756x; 1.0756x over previous
import functools

import jax
import jax.numpy as jnp
import numpy as np
from jax import lax
from jax.experimental import pallas as pl
from jax.experimental.pallas import tpu as pltpu
from jax.experimental.pallas import tpu_sc as plsc

_USE_TEXT_QUERY = True
_USE_TEXT_CANDS = True
_N_CANDS = 5
_TEXT_QUESTION = 0
_TEXT_EMBEDDING = 1
_VISUAL_EMBEDDING = 3
_PREFIX_ROWS = 16


@functools.lru_cache(maxsize=None)
def _make_sc_prefix(D: int):
    mesh = plsc.ScalarSubcoreMesh(axis_name="c", num_cores=1)
    n_text = (1 if _USE_TEXT_QUERY else 0) + (
        _N_CANDS if _USE_TEXT_CANDS else 0
    )
    ids = [_TEXT_QUESTION] + [_TEXT_EMBEDDING] * (n_text - 1)
    ids += [_VISUAL_EMBEDDING] * (_PREFIX_ROWS - len(ids))

    @functools.partial(
        pl.kernel,
        mesh=mesh,
        out_type=jax.ShapeDtypeStruct((_PREFIX_ROWS, D), jnp.float32),
        scratch_types=[pltpu.SemaphoreType.DMA],
    )
    def k(table_hbm, out_hbm, sem):
        copies = [
            pltpu.async_copy(
                table_hbm.at[pl.ds(t, 1)], out_hbm.at[pl.ds(r, 1)], sem
            )
            for r, t in enumerate(ids)
        ]
        for c in copies:
            c.wait()

    return k


@functools.lru_cache(maxsize=None)
def _make_tc_fill(L: int, D: int):
    BLK = 1024
    assert L % BLK == 0

    def body(table_ref, out_ref):
        row = table_ref[_VISUAL_EMBEDDING, :]
        out_ref[...] = jnp.broadcast_to(row[None, None, :], (BLK, 1, D))

    return pl.pallas_call(
        body,
        grid=(L // BLK,),
        in_specs=[pl.BlockSpec((5, D), lambda i: (0, 0))],
        out_specs=pl.BlockSpec((BLK, 1, D), lambda i: (i, 0, 0)),
        out_shape=jax.ShapeDtypeStruct((L, 1, D), jnp.float32),
    )


@functools.lru_cache(maxsize=None)
def _make_tc_patch(L: int, D: int):
    def body(full_ref, prefix_ref, out_ref):
        out_ref[...] = prefix_ref[...][:, None, :]

    return pl.pallas_call(
        body,
        grid=(1,),
        in_specs=[
            pl.BlockSpec((_PREFIX_ROWS, 1, D), lambda i: (0, 0, 0)),
            pl.BlockSpec((_PREFIX_ROWS, D), lambda i: (0, 0)),
        ],
        out_specs=pl.BlockSpec((_PREFIX_ROWS, 1, D), lambda i: (0, 0, 0)),
        out_shape=jax.ShapeDtypeStruct((L, 1, D), jnp.float32),
        input_output_aliases={0: 0},
    )


def kernel(x, table):
    L, N, D = x.shape
    prefix = _make_sc_prefix(D)(table)
    full = _make_tc_fill(L, D)(table)
    return _make_tc_patch(L, D)(full, prefix)


# speedup vs baseline: 1.0761x; 1.0005x over previous
import functools

import jax
import jax.numpy as jnp
from jax.experimental import pallas as pl
from jax.experimental.pallas import tpu as pltpu
from jax.experimental.pallas import tpu_sc as plsc

_USE_TEXT_QUERY = True
_USE_TEXT_CANDS = True
_N_CANDS = 5
_TEXT_QUESTION = 0
_TEXT_EMBEDDING = 1
_VISUAL_EMBEDDING = 3
_PREFIX_ROWS = 16


@functools.lru_cache(maxsize=None)
def _make_sc_prefix(D: int):
    mesh = plsc.ScalarSubcoreMesh(axis_name="c", num_cores=1)
    n_text = (1 if _USE_TEXT_QUERY else 0) + (
        _N_CANDS if _USE_TEXT_CANDS else 0
    )
    ids = [_TEXT_QUESTION] + [_TEXT_EMBEDDING] * (n_text - 1)
    ids += [_VISUAL_EMBEDDING] * (_PREFIX_ROWS - len(ids))

    @functools.partial(
        pl.kernel,
        mesh=mesh,
        out_type=jax.ShapeDtypeStruct((_PREFIX_ROWS, D), jnp.float32),
        scratch_types=[pltpu.SemaphoreType.DMA],
    )
    def k(table_hbm, out_hbm, sem):
        copies = [
            pltpu.async_copy(
                table_hbm.at[pl.ds(t, 1)], out_hbm.at[pl.ds(r, 1)], sem
            )
            for r, t in enumerate(ids)
        ]
        for c in copies:
            c.wait()

    return k


@functools.lru_cache(maxsize=None)
def _make_tc_fill(L: int, D: int):
    BLK = 1024
    assert L % BLK == 0

    def body(table_ref, out_ref):
        row = table_ref[_VISUAL_EMBEDDING, :]
        out_ref[...] = jnp.broadcast_to(row[None, None, :], (BLK, 1, D))

    return pl.pallas_call(
        body,
        grid=(L // BLK,),
        in_specs=[pl.BlockSpec((5, D), lambda i: (0, 0))],
        out_specs=pl.BlockSpec((BLK, 1, D), lambda i: (i, 0, 0)),
        out_shape=jax.ShapeDtypeStruct((L, 1, D), jnp.float32),
    )


@functools.lru_cache(maxsize=None)
def _make_tc_patch(L: int, D: int):
    def body(full_ref, prefix_ref, out_ref):
        out_ref[...] = prefix_ref[...][:, None, :]

    return pl.pallas_call(
        body,
        grid=(1,),
        in_specs=[
            pl.BlockSpec((_PREFIX_ROWS, 1, D), lambda i: (0, 0, 0)),
            pl.BlockSpec((_PREFIX_ROWS, D), lambda i: (0, 0)),
        ],
        out_specs=pl.BlockSpec((_PREFIX_ROWS, 1, D), lambda i: (0, 0, 0)),
        out_shape=jax.ShapeDtypeStruct((L, 1, D), jnp.float32),
        input_output_aliases={0: 0},
    )


def kernel(x, table):
    L, N, D = x.shape
    full = _make_tc_fill(L, D)(table)
    prefix = _make_sc_prefix(D)(table)
    return _make_tc_patch(L, D)(full, prefix)
